# baseline (speedup 1.0000x reference)
import functools

import jax
import jax.numpy as jnp
from jax import lax
from jax.experimental import pallas as pl
from jax.experimental.pallas import tpu as pltpu
from jax.experimental.pallas import tpu_sc as plsc


def _embed_concat(X_num, X_cat_T, tab_flat, *, B, F, V, D, NN):
    OUT_W = F * D + NN
    info = plsc.get_sparse_core_info()
    NC, NS = info.num_cores, info.num_subcores
    NW = NC * NS
    BW = B // NW
    CH = 128
    NCH = BW // CH

    mesh = plsc.VectorSubcoreMesh(core_axis_name="c", subcore_axis_name="s")

    @functools.partial(
        pl.kernel,
        out_type=jax.ShapeDtypeStruct((B, OUT_W), jnp.float32),
        mesh=mesh,
        compiler_params=pltpu.CompilerParams(use_tc_tiling_on_sc=False),
        scratch_types=[
            pltpu.VMEM((BW,), jnp.int32),
            pltpu.VMEM((BW, D), jnp.float32),
            pltpu.VMEM((BW, NN), jnp.float32),
            pltpu.SemaphoreType.DMA,
        ],
    )
    def run(xcat_hbm, xnum_hbm, tab_hbm, out_hbm,
            idx_v, emb_v, xnum_v, sem):
        wid = lax.axis_index("s") * NC + lax.axis_index("c")
        wbase = pl.multiple_of(wid * BW, BW)
        pltpu.sync_copy(xnum_hbm.at[pl.ds(wbase, BW)], xnum_v)
        pltpu.sync_copy(xnum_v, out_hbm.at[pl.ds(wbase, BW), pl.ds(F * D, NN)])
        for f in range(F):
            pltpu.sync_copy(xcat_hbm.at[f, pl.ds(wbase, BW)], idx_v)
            for k in range(BW // 16):
                sl = pl.ds(k * 16, 16)
                idx_v[sl] = idx_v[sl] + (f * V)
            cps = [
                pltpu.async_copy(
                    tab_hbm.at[idx_v.at[pl.ds(c * CH, CH)]],
                    emb_v.at[pl.ds(c * CH, CH)],
                    sem,
                )
                for c in range(NCH)
            ]
            for cp in cps:
                cp.wait()
            pltpu.sync_copy(
                emb_v, out_hbm.at[pl.ds(wbase, BW), pl.ds(f * D, D)]
            )

    return run(X_cat_T, X_num, tab_flat)


def kernel(X_num, X_cat, tables):
    B, NN = X_num.shape
    _, F = X_cat.shape
    _, V, D = tables.shape
    xcat_T = X_cat.astype(jnp.int32).T
    tab_flat = tables.reshape(F * V, D)
    return _embed_concat(X_num, xcat_T, tab_flat, B=B, F=F, V=V, D=D, NN=NN)


# speedup vs baseline: 1.0265x; 1.0265x over previous
import functools

import jax
import jax.numpy as jnp
from jax import lax
from jax.experimental import pallas as pl
from jax.experimental.pallas import tpu as pltpu
from jax.experimental.pallas import tpu_sc as plsc

_NBUF = 4


def _embed_concat(X_num, X_cat_T, tab_flat, *, B, F, V, D, NN):
    OUT_W = F * D + NN
    info = plsc.get_sparse_core_info()
    NC, NS = info.num_cores, info.num_subcores
    NW = NC * NS
    BW = B // NW
    CH = 128
    NCH = BW // CH

    mesh = plsc.VectorSubcoreMesh(core_axis_name="c", subcore_axis_name="s")

    @functools.partial(
        pl.kernel,
        out_type=jax.ShapeDtypeStruct((B, OUT_W), jnp.float32),
        mesh=mesh,
        compiler_params=pltpu.CompilerParams(use_tc_tiling_on_sc=False),
        scratch_types=[
            pltpu.VMEM((F, BW), jnp.int32),
            [pltpu.VMEM((BW, D), jnp.float32)] * _NBUF,
            pltpu.VMEM((BW, NN), jnp.float32),
            [pltpu.SemaphoreType.DMA] * _NBUF,
            [pltpu.SemaphoreType.DMA] * _NBUF,
            pltpu.SemaphoreType.DMA,
            pltpu.SemaphoreType.DMA,
        ],
    )
    def run(xcat_hbm, xnum_hbm, tab_hbm, out_hbm,
            idx2, embs, xnum_v, gsems, wsems, isem, nsem):
        wid = lax.axis_index("s") * NC + lax.axis_index("c")
        wbase = pl.multiple_of(wid * BW, BW)

        def gather(f):
            b = f % _NBUF
            return [
                pltpu.async_copy(
                    tab_hbm.at[pl.ds(f * V, V)]
                    .at[idx2.at[f, pl.ds(c * CH, CH)]],
                    embs[b].at[pl.ds(c * CH, CH)],
                    gsems[b],
                )
                for c in range(NCH)
            ]

        def write(f):
            b = f % _NBUF
            return pltpu.async_copy(
                embs[b], out_hbm.at[pl.ds(wbase, BW), pl.ds(f * D, D)],
                wsems[b],
            )

        icp = pltpu.async_copy(xcat_hbm.at[:, pl.ds(wbase, BW)], idx2, isem)
        ncp = pltpu.async_copy(xnum_hbm.at[pl.ds(wbase, BW)], xnum_v, nsem)
        icp.wait()

        g, w = {}, {}
        for f in range(_NBUF - 1):
            g[f] = gather(f)
        ncp.wait()
        nw = pltpu.async_copy(
            xnum_v, out_hbm.at[pl.ds(wbase, BW), pl.ds(F * D, NN)], nsem
        )
        for f in range(F):
            for cp in g[f]:
                cp.wait()
            w[f] = write(f)
            nf = f + _NBUF - 1
            if nf < F:
                if f >= 1:
                    w[f - 1].wait()
                g[nf] = gather(nf)
        for f in range(F - _NBUF, F):
            w[f].wait()
        nw.wait()

    return run(X_cat_T, X_num, tab_flat)


def kernel(X_num, X_cat, tables):
    B, NN = X_num.shape
    _, F = X_cat.shape
    _, V, D = tables.shape
    xcat_T = X_cat.astype(jnp.int32).T
    tab_flat = tables.reshape(F * V, D)
    return _embed_concat(X_num, xcat_T, tab_flat, B=B, F=F, V=V, D=D, NN=NN)
